# baseline (speedup 1.0000x reference)
import functools

import jax
import jax.numpy as jnp
from jax import lax
from jax.experimental import pallas as pl
from jax.experimental.pallas import tpu as pltpu
from jax.experimental.pallas import tpu_sc as plsc

_VIEW = 2
_N = 2048
_IN = 1024
_FD = 512
_HD = 128
_CN = 10
_HID = 1800
_K = 10

_BM = 256


_H = 1792


def _dist_body(xb_ref, xa_ref, d_ref, *, bm, n, off):
    xb = xb_ref[0]
    xa = xa_ref[0]
    d = -2.0 * jax.lax.dot_general(
        xb, xa, (((1,), (1,)), ((), ())), preferred_element_type=jnp.float32)
    cn = jnp.sum(xa * xa, axis=1)
    d = d + cn[None, :]
    i = pl.program_id(1)
    rows = off + i * bm + jax.lax.broadcasted_iota(jnp.int32, (bm, n), 0)
    cols = jax.lax.broadcasted_iota(jnp.int32, (bm, n), 1)
    d_ref[0] = jnp.where(cols == rows, jnp.float32(3.0e38), d)


def _dist_sc(xs):
    nb = (_N - _H) // _BM
    hb = _H // _BM
    body = functools.partial(_dist_body, bm=_BM, n=_N, off=_H)
    return pl.pallas_call(
        body,
        grid=(_VIEW, nb),
        in_specs=[
            pl.BlockSpec((1, _BM, _IN), lambda v, i: (v, hb + i, 0)),
            pl.BlockSpec((1, _N, _IN), lambda v, i: (v, 0, 0)),
        ],
        out_specs=pl.BlockSpec((1, _BM, _N), lambda v, i: (v, i, 0)),
        out_shape=jax.ShapeDtypeStruct((_VIEW, _N - _H, _N), jnp.float32),
    )(xs, xs)


def _knn_tc_body(xb_ref, xa_ref, g_ref, *, bm, n, k):
    xb = xb_ref[0]
    xa = xa_ref[0]
    d = -2.0 * jax.lax.dot_general(
        xb, xa, (((1,), (1,)), ((), ())), preferred_element_type=jnp.float32)
    cn = jnp.sum(xa * xa, axis=1)
    d = d + cn[None, :]
    i = pl.program_id(1)
    rows = i * bm + jax.lax.broadcasted_iota(jnp.int32, (bm, n), 0)
    cols = jax.lax.broadcasted_iota(jnp.int32, (bm, n), 1)
    big = jnp.float32(jnp.inf)
    d = jnp.where(cols == rows, big, d)
    g = jnp.zeros((bm, n), jnp.float32)
    for _ in range(k - 1):
        m = jnp.min(d, axis=1, keepdims=True)
        eq = d == m
        first = jnp.min(jnp.where(eq, cols, n), axis=1)
        oh = cols == first[:, None]
        g = jnp.where(oh, 1.0, g)
        d = jnp.where(oh, big, d)
    g_ref[0] = g


def _knn_tc(xs):
    nb = _H // _BM
    body = functools.partial(_knn_tc_body, bm=_BM, n=_N, k=_K)
    return pl.pallas_call(
        body,
        grid=(_VIEW, nb),
        in_specs=[
            pl.BlockSpec((1, _BM, _IN), lambda v, i: (v, i, 0)),
            pl.BlockSpec((1, _N, _IN), lambda v, i: (v, 0, 0)),
        ],
        out_specs=pl.BlockSpec((1, _BM, _N), lambda v, i: (v, i, 0)),
        out_shape=jax.ShapeDtypeStruct((_VIEW, _H, _N), jnp.float32),
    )(xs, xs)


_NC = 2
_NW = 32
_RB = 16


def _sc_topk(d2):
    rows_total, n = d2.shape
    rpw = rows_total // _NW
    nrb = rpw // _RB
    nblk = n // 128
    mesh = plsc.VectorSubcoreMesh(core_axis_name="c", subcore_axis_name="s")
    big = jnp.float32(3.3e38)

    @functools.partial(
        pl.kernel, mesh=mesh,
        compiler_params=pltpu.CompilerParams(needs_layout_passes=False),
        out_type=jax.ShapeDtypeStruct((rows_total, n), jnp.float32),
        scratch_types=[
            pltpu.VMEM((_RB, n), jnp.float32),
            pltpu.VMEM((_RB, n), jnp.float32),
            pltpu.VMEM((_RB, 16), jnp.int32),
        ],
    )
    def k(d_hbm, g_hbm, in_v, oh_v, bi_v):
        wid = lax.axis_index("s") * _NC + lax.axis_index("c")
        base = wid * rpw
        lane = lax.iota(jnp.int32, 16)
        zeros16 = jnp.zeros((16,), jnp.float32)
        ones9 = jnp.where(lane < (_K - 1), 1.0, 0.0).astype(jnp.float32)
        mask9 = lane < (_K - 1)

        def zrow(c, carry):
            oh_v[c // (n // 16), pl.ds((c % (n // 16)) * 16, 16)] = zeros16
            return carry
        lax.fori_loop(0, _RB * (n // 16), zrow, 0)

        def row_block(rb, carry):
            r0 = base + rb * _RB
            pltpu.sync_copy(d_hbm.at[pl.ds(r0, _RB)], in_v)

            idx8 = jnp.full((16, 1), _K - 2, jnp.int32)
            gdn = lax.GatherDimensionNumbers(
                offset_dims=(), collapsed_slice_dims=(0,),
                start_index_map=(0,))

            def one_row(j, carry2):
                def blk(b, st):
                    bv, bi = st
                    thr = lax.gather(
                        bv, idx8, gdn, (1,),
                        mode=lax.GatherScatterMode.PROMISE_IN_BOUNDS)

                    def ld(t):
                        return in_v[j, pl.ds((b * 8 + t) * 16, 16)]

                    mall = jnp.minimum(
                        jnp.minimum(jnp.minimum(ld(0), ld(1)),
                                    jnp.minimum(ld(2), ld(3))),
                        jnp.minimum(jnp.minimum(ld(4), ld(5)),
                                    jnp.minimum(ld(6), ld(7))))
                    hit = lax.reduce_or_p.bind(mall < thr, axes=(0,))

                    def do_merge(bv, bi):
                        for t in range(8):
                            def merge_one(bv, bi, t=t):
                                v = ld(t)
                                ii = (b * 8 + t) * 16 + lane
                                sv, si = plsc.sort_key_val(v, ii)
                                rv = lax.rev(sv, (0,))
                                ri = lax.rev(si, (0,))
                                m = rv < bv
                                bv2, bi2 = plsc.sort_key_val(
                                    jnp.where(m, rv, bv),
                                    jnp.where(m, ri, bi))
                                return bv2, bi2

                            chit = lax.reduce_or_p.bind(
                                ld(t) < thr, axes=(0,))
                            bv, bi = lax.cond(
                                chit, merge_one, lambda a, c: (a, c), bv, bi)
                        return bv, bi

                    return lax.cond(hit, do_merge,
                                    lambda a, c: (a, c), bv, bi)

                bv0 = jnp.full((16,), big, jnp.float32)
                bi0 = jnp.zeros((16,), jnp.int32)
                bv, bi = lax.fori_loop(0, nblk, blk, (bv0, bi0))
                rowv = jnp.full((16,), j, jnp.int32)
                plsc.store_scatter(oh_v, [rowv, bi], ones9, mask=mask9)
                bi_v[j] = bi
                return carry2

            lax.fori_loop(0, _RB, one_row, 0)
            pltpu.sync_copy(oh_v, g_hbm.at[pl.ds(r0, _RB)])

            def unset(j, carry2):
                rowv = jnp.full((16,), j, jnp.int32)
                plsc.store_scatter(oh_v, [rowv, bi_v[j]], zeros16, mask=mask9)
                return carry2

            lax.fori_loop(0, _RB, unset, 0)
            return carry

        lax.fori_loop(0, nrb, row_block, 0)

    return k(d2)


def _knn(xs):
    d = _dist_sc(xs)
    g_sc = _sc_topk(d.reshape(_VIEW * (_N - _H), _N))
    g_tc = _knn_tc(xs)
    return jnp.concatenate(
        [g_tc, g_sc.reshape(_VIEW, _N - _H, _N)], axis=1)


def _sym_body(gr_ref, gc_ref, g_ref, deg_ref):
    g = jnp.maximum(gr_ref[0], gc_ref[0].T)
    g_ref[0] = g
    deg_ref[0, 0, 0, :] = jnp.sum(g, axis=1) + 1.0


def _sym(g0):
    nb = _N // _BM
    gs, deg = pl.pallas_call(
        _sym_body,
        grid=(_VIEW, nb),
        in_specs=[
            pl.BlockSpec((1, _BM, _N), lambda v, i: (v, i, 0)),
            pl.BlockSpec((1, _N, _BM), lambda v, i: (v, 0, i)),
        ],
        out_specs=[
            pl.BlockSpec((1, _BM, _N), lambda v, i: (v, i, 0)),
            pl.BlockSpec((1, 1, 1, _BM), lambda v, i: (v, i, 0, 0)),
        ],
        out_shape=[
            jax.ShapeDtypeStruct((_VIEW, _N, _N), jnp.float32),
            jax.ShapeDtypeStruct((_VIEW, nb, 1, _BM), jnp.float32),
        ],
    )(g0, g0)
    return gs, deg.reshape(_VIEW, 1, _N)


def _scale_body(g_ref, deg_ref, s_ref, *, bm, n):
    i = pl.program_id(1)
    g = g_ref[0]
    rows = i * bm + jax.lax.broadcasted_iota(jnp.int32, (bm, n), 0)
    cols = jax.lax.broadcasted_iota(jnp.int32, (bm, n), 1)
    g = jnp.where(cols == rows, g + 1.0, g)
    dinv_c = jax.lax.rsqrt(deg_ref[0, 0, :])
    dinv_r = jax.lax.rsqrt(deg_ref[0, 0, pl.ds(i * bm, bm)])
    s_ref[0] = (g * dinv_r[:, None] * dinv_c[None, :]).astype(jnp.bfloat16)


def _scale(gs, deg):
    nb = _N // _BM
    body = functools.partial(_scale_body, bm=_BM, n=_N)
    return pl.pallas_call(
        body,
        grid=(_VIEW, nb),
        in_specs=[
            pl.BlockSpec((1, _BM, _N), lambda v, i: (v, i, 0)),
            pl.BlockSpec((1, 1, _N), lambda v, i: (v, 0, 0)),
        ],
        out_specs=pl.BlockSpec((1, _BM, _N), lambda v, i: (v, i, 0)),
        out_shape=jax.ShapeDtypeStruct((_VIEW, _N, _N), jnp.bfloat16),
    )(gs, deg)


def _act_linear(x):
    return x


def _act_tanh(x):
    return jnp.tanh(x)


def _act_sigmoid(x):
    return jax.nn.sigmoid(x)


def _act_l2n(x):
    nrm = jnp.sqrt(jnp.sum(x * x, axis=1, keepdims=True))
    return x / jnp.maximum(nrm, 1e-12)


def _act_softmax(x):
    m = jnp.max(x, axis=1, keepdims=True)
    e = jnp.exp(x - m)
    return e / jnp.sum(e, axis=1, keepdims=True)


def _mm_body(a_ref, b_ref, o_ref, *, act, trans_b):
    a = a_ref[0].astype(jnp.bfloat16)
    b = b_ref[0].astype(jnp.bfloat16)
    if trans_b:
        r = jax.lax.dot_general(a, b, (((1,), (1,)), ((), ())),
                                preferred_element_type=jnp.float32)
    else:
        r = jnp.dot(a, b, preferred_element_type=jnp.float32)
    o_ref[0] = act(r).astype(o_ref.dtype)


def _mmb_body(a_ref, b_ref, bias_ref, o_ref, *, act, trans_b):
    a = a_ref[0].astype(jnp.bfloat16)
    b = b_ref[0].astype(jnp.bfloat16)
    r = jnp.dot(a, b, preferred_element_type=jnp.float32)
    o_ref[0] = act(r + bias_ref[0, 0:1, :]).astype(o_ref.dtype)


def _pick_bn(np_):
    for c in (512, 384, 256, 128):
        if np_ % c == 0:
            return c
    return np_


def _mm(a, b, *, act=_act_linear, bias=None, trans_b=False, bn=None,
        out_dtype=jnp.bfloat16):
    _, m, k = a.shape
    b_batched = b.ndim == 3
    bshape = b.shape[1:] if b_batched else b.shape
    np_ = bshape[0] if trans_b else bshape[1]
    if bn is None:
        bn = _pick_bn(np_)
    grid = (_VIEW, np_ // bn)
    in_specs = [pl.BlockSpec((1, m, k), lambda v, j: (v, 0, 0))]
    bsel = (lambda v: v) if b_batched else (lambda v: 0)
    if not b_batched:
        b = b[None]
    if trans_b:
        in_specs.append(pl.BlockSpec((1, bn, k), lambda v, j: (bsel(v), j, 0)))
    else:
        in_specs.append(pl.BlockSpec((1, k, bn), lambda v, j: (bsel(v), 0, j)))
    args = [a, b]
    if bias is not None:
        bias_batched = bias.ndim == 2
        if not bias_batched:
            bias = bias[None]
        bias = bias[:, None, :]
        bisel = (lambda v: v) if bias_batched else (lambda v: 0)
        in_specs.append(
            pl.BlockSpec((1, 1, bn), lambda v, j: (bisel(v), 0, j)))
        args.append(bias)
        body = functools.partial(_mmb_body, act=act, trans_b=trans_b)
    else:
        body = functools.partial(_mm_body, act=act, trans_b=trans_b)
    return pl.pallas_call(
        body,
        grid=grid,
        in_specs=in_specs,
        out_specs=pl.BlockSpec((1, m, bn), lambda v, j: (v, 0, j)),
        out_shape=jax.ShapeDtypeStruct((_VIEW, m, np_), out_dtype),
    )(*args)


def kernel(xs, enc_w1, enc_b1, enc_w2, enc_b2, enc_wz,
           dec_w0, dec_w1, dec_b1, dec_w2, dec_b2,
           fcm_w, fcm_b, lcm_w, lcm_b):
    f32 = jnp.float32
    g0 = _knn(xs)
    gs, deg = _sym(g0)
    s = _scale(gs, deg)

    sx = _mm(s, xs)
    o1 = _mm(sx, enc_w1, bias=enc_b1, act=_act_tanh, bn=_HID)
    so1 = _mm(s, o1, bn=_HID)
    z = _mm(so1, enc_w2, bias=enc_b2, act=_act_tanh, out_dtype=f32)
    zb = z.astype(jnp.bfloat16)

    zwz = _mm(zb, enc_wz)
    ars = _mm(zwz, zb, trans_b=True, act=_act_sigmoid, out_dtype=f32)

    h = _mm(zb, fcm_w, bias=fcm_b, act=_act_l2n, bn=_HD, out_dtype=f32)
    q = _mm(zb, lcm_w, bias=lcm_b, act=_act_softmax, bn=_CN, out_dtype=f32)

    h1 = _mm(zb, dec_w0, act=_act_tanh)
    sh1 = _mm(s, h1)
    h11 = _mm(sh1, dec_w1, bias=dec_b1, act=_act_tanh, bn=_HID)
    sh11 = _mm(s, h11, bn=_HID)
    xr = _mm(sh11, dec_w2, bias=dec_b2, act=_act_tanh, out_dtype=f32)

    return h, q, xr, z, gs, ars


# speedup vs baseline: 1.0152x; 1.0152x over previous
import functools

import jax
import jax.numpy as jnp
from jax import lax
from jax.experimental import pallas as pl
from jax.experimental.pallas import tpu as pltpu
from jax.experimental.pallas import tpu_sc as plsc

_VIEW = 2
_N = 2048
_IN = 1024
_FD = 512
_HD = 128
_CN = 10
_HID = 1800
_K = 10

_BM = 256


_H = 1536


def _dist_body(xb_ref, xa_ref, d_ref, *, bm, n, off):
    xb = xb_ref[0]
    xa = xa_ref[0]
    d = -2.0 * jax.lax.dot_general(
        xb, xa, (((1,), (1,)), ((), ())), preferred_element_type=jnp.float32)
    cn = jnp.sum(xa * xa, axis=1)
    d = d + cn[None, :]
    i = pl.program_id(1)
    rows = off + i * bm + jax.lax.broadcasted_iota(jnp.int32, (bm, n), 0)
    cols = jax.lax.broadcasted_iota(jnp.int32, (bm, n), 1)
    d_ref[0] = jnp.where(cols == rows, jnp.float32(3.0e38), d)


def _dist_sc(xs):
    nb = (_N - _H) // _BM
    hb = _H // _BM
    body = functools.partial(_dist_body, bm=_BM, n=_N, off=_H)
    return pl.pallas_call(
        body,
        grid=(_VIEW, nb),
        in_specs=[
            pl.BlockSpec((1, _BM, _IN), lambda v, i: (v, hb + i, 0)),
            pl.BlockSpec((1, _N, _IN), lambda v, i: (v, 0, 0)),
        ],
        out_specs=pl.BlockSpec((1, _BM, _N), lambda v, i: (v, i, 0)),
        out_shape=jax.ShapeDtypeStruct((_VIEW, _N - _H, _N), jnp.float32),
    )(xs, xs)


def _knn_tc_body(xb_ref, xa_ref, g_ref, *, bm, n, k):
    xb = xb_ref[0]
    xa = xa_ref[0]
    d = -2.0 * jax.lax.dot_general(
        xb, xa, (((1,), (1,)), ((), ())), preferred_element_type=jnp.float32)
    cn = jnp.sum(xa * xa, axis=1)
    d = d + cn[None, :]
    i = pl.program_id(1)
    rows = i * bm + jax.lax.broadcasted_iota(jnp.int32, (bm, n), 0)
    cols = jax.lax.broadcasted_iota(jnp.int32, (bm, n), 1)
    big = jnp.float32(jnp.inf)
    d = jnp.where(cols == rows, big, d)
    g = jnp.zeros((bm, n), jnp.float32)
    for _ in range(k - 1):
        m = jnp.min(d, axis=1, keepdims=True)
        eq = d == m
        first = jnp.min(jnp.where(eq, cols, n), axis=1)
        oh = cols == first[:, None]
        g = jnp.where(oh, 1.0, g)
        d = jnp.where(oh, big, d)
    g_ref[0] = g


def _knn_tc(xs):
    nb = _H // _BM
    body = functools.partial(_knn_tc_body, bm=_BM, n=_N, k=_K)
    return pl.pallas_call(
        body,
        grid=(_VIEW, nb),
        in_specs=[
            pl.BlockSpec((1, _BM, _IN), lambda v, i: (v, i, 0)),
            pl.BlockSpec((1, _N, _IN), lambda v, i: (v, 0, 0)),
        ],
        out_specs=pl.BlockSpec((1, _BM, _N), lambda v, i: (v, i, 0)),
        out_shape=jax.ShapeDtypeStruct((_VIEW, _H, _N), jnp.float32),
    )(xs, xs)


_NC = 2
_NW = 32
_RB = 16


def _sc_topk(d2):
    rows_total, n = d2.shape
    rpw = rows_total // _NW
    nrb = rpw // _RB
    nblk = n // 128
    mesh = plsc.VectorSubcoreMesh(core_axis_name="c", subcore_axis_name="s")
    big = jnp.float32(3.3e38)

    @functools.partial(
        pl.kernel, mesh=mesh,
        compiler_params=pltpu.CompilerParams(needs_layout_passes=False),
        out_type=jax.ShapeDtypeStruct((rows_total, n), jnp.float32),
        scratch_types=[
            pltpu.VMEM((_RB, n), jnp.float32),
            pltpu.VMEM((_RB, n), jnp.float32),
            pltpu.VMEM((_RB, 16), jnp.int32),
        ],
    )
    def k(d_hbm, g_hbm, in_v, oh_v, bi_v):
        wid = lax.axis_index("s") * _NC + lax.axis_index("c")
        base = wid * rpw
        lane = lax.iota(jnp.int32, 16)
        zeros16 = jnp.zeros((16,), jnp.float32)
        ones9 = jnp.where(lane < (_K - 1), 1.0, 0.0).astype(jnp.float32)
        mask9 = lane < (_K - 1)

        def zrow(c, carry):
            oh_v[c // (n // 16), pl.ds((c % (n // 16)) * 16, 16)] = zeros16
            return carry
        lax.fori_loop(0, _RB * (n // 16), zrow, 0)

        def row_block(rb, carry):
            r0 = base + rb * _RB
            pltpu.sync_copy(d_hbm.at[pl.ds(r0, _RB)], in_v)

            idx8 = jnp.full((16, 1), _K - 2, jnp.int32)
            gdn = lax.GatherDimensionNumbers(
                offset_dims=(), collapsed_slice_dims=(0,),
                start_index_map=(0,))

            def one_row(j, carry2):
                def blk(b, st):
                    bv, bi = st
                    thr = lax.gather(
                        bv, idx8, gdn, (1,),
                        mode=lax.GatherScatterMode.PROMISE_IN_BOUNDS)

                    def ld(t):
                        return in_v[j, pl.ds((b * 8 + t) * 16, 16)]

                    mall = jnp.minimum(
                        jnp.minimum(jnp.minimum(ld(0), ld(1)),
                                    jnp.minimum(ld(2), ld(3))),
                        jnp.minimum(jnp.minimum(ld(4), ld(5)),
                                    jnp.minimum(ld(6), ld(7))))
                    hit = lax.reduce_or_p.bind(mall < thr, axes=(0,))

                    def do_merge(bv, bi):
                        for t in range(8):
                            def merge_one(bv, bi, t=t):
                                v = ld(t)
                                ii = (b * 8 + t) * 16 + lane
                                sv, si = plsc.sort_key_val(v, ii)
                                rv = lax.rev(sv, (0,))
                                ri = lax.rev(si, (0,))
                                m = rv < bv
                                bv2, bi2 = plsc.sort_key_val(
                                    jnp.where(m, rv, bv),
                                    jnp.where(m, ri, bi))
                                return bv2, bi2

                            chit = lax.reduce_or_p.bind(
                                ld(t) < thr, axes=(0,))
                            bv, bi = lax.cond(
                                chit, merge_one, lambda a, c: (a, c), bv, bi)
                        return bv, bi

                    return lax.cond(hit, do_merge,
                                    lambda a, c: (a, c), bv, bi)

                bv0 = jnp.full((16,), big, jnp.float32)
                bi0 = jnp.zeros((16,), jnp.int32)
                bv, bi = lax.fori_loop(0, nblk, blk, (bv0, bi0))
                rowv = jnp.full((16,), j, jnp.int32)
                plsc.store_scatter(oh_v, [rowv, bi], ones9, mask=mask9)
                bi_v[j] = bi
                return carry2

            lax.fori_loop(0, _RB, one_row, 0)
            pltpu.sync_copy(oh_v, g_hbm.at[pl.ds(r0, _RB)])

            def unset(j, carry2):
                rowv = jnp.full((16,), j, jnp.int32)
                plsc.store_scatter(oh_v, [rowv, bi_v[j]], zeros16, mask=mask9)
                return carry2

            lax.fori_loop(0, _RB, unset, 0)
            return carry

        lax.fori_loop(0, nrb, row_block, 0)

    return k(d2)


def _knn(xs):
    d = _dist_sc(xs)
    g_sc = _sc_topk(d.reshape(_VIEW * (_N - _H), _N))
    g_tc = _knn_tc(xs)
    return jnp.concatenate(
        [g_tc, g_sc.reshape(_VIEW, _N - _H, _N)], axis=1)


def _sym_body(gr_ref, gc_ref, g_ref, deg_ref):
    g = jnp.maximum(gr_ref[0], gc_ref[0].T)
    g_ref[0] = g
    deg_ref[0, 0, 0, :] = jnp.sum(g, axis=1) + 1.0


def _sym(g0):
    nb = _N // _BM
    gs, deg = pl.pallas_call(
        _sym_body,
        grid=(_VIEW, nb),
        in_specs=[
            pl.BlockSpec((1, _BM, _N), lambda v, i: (v, i, 0)),
            pl.BlockSpec((1, _N, _BM), lambda v, i: (v, 0, i)),
        ],
        out_specs=[
            pl.BlockSpec((1, _BM, _N), lambda v, i: (v, i, 0)),
            pl.BlockSpec((1, 1, 1, _BM), lambda v, i: (v, i, 0, 0)),
        ],
        out_shape=[
            jax.ShapeDtypeStruct((_VIEW, _N, _N), jnp.float32),
            jax.ShapeDtypeStruct((_VIEW, nb, 1, _BM), jnp.float32),
        ],
    )(g0, g0)
    return gs, deg.reshape(_VIEW, 1, _N)


def _scale_body(g_ref, deg_ref, s_ref, *, bm, n):
    i = pl.program_id(1)
    g = g_ref[0]
    rows = i * bm + jax.lax.broadcasted_iota(jnp.int32, (bm, n), 0)
    cols = jax.lax.broadcasted_iota(jnp.int32, (bm, n), 1)
    g = jnp.where(cols == rows, g + 1.0, g)
    dinv_c = jax.lax.rsqrt(deg_ref[0, 0, :])
    dinv_r = jax.lax.rsqrt(deg_ref[0, 0, pl.ds(i * bm, bm)])
    s_ref[0] = (g * dinv_r[:, None] * dinv_c[None, :]).astype(jnp.bfloat16)


def _scale(gs, deg):
    nb = _N // _BM
    body = functools.partial(_scale_body, bm=_BM, n=_N)
    return pl.pallas_call(
        body,
        grid=(_VIEW, nb),
        in_specs=[
            pl.BlockSpec((1, _BM, _N), lambda v, i: (v, i, 0)),
            pl.BlockSpec((1, 1, _N), lambda v, i: (v, 0, 0)),
        ],
        out_specs=pl.BlockSpec((1, _BM, _N), lambda v, i: (v, i, 0)),
        out_shape=jax.ShapeDtypeStruct((_VIEW, _N, _N), jnp.bfloat16),
    )(gs, deg)


def _act_linear(x):
    return x


def _act_tanh(x):
    return jnp.tanh(x)


def _act_sigmoid(x):
    return jax.nn.sigmoid(x)


def _act_l2n(x):
    nrm = jnp.sqrt(jnp.sum(x * x, axis=1, keepdims=True))
    return x / jnp.maximum(nrm, 1e-12)


def _act_softmax(x):
    m = jnp.max(x, axis=1, keepdims=True)
    e = jnp.exp(x - m)
    return e / jnp.sum(e, axis=1, keepdims=True)


def _mm_body(a_ref, b_ref, o_ref, *, act, trans_b):
    a = a_ref[0].astype(jnp.bfloat16)
    b = b_ref[0].astype(jnp.bfloat16)
    if trans_b:
        r = jax.lax.dot_general(a, b, (((1,), (1,)), ((), ())),
                                preferred_element_type=jnp.float32)
    else:
        r = jnp.dot(a, b, preferred_element_type=jnp.float32)
    o_ref[0] = act(r).astype(o_ref.dtype)


def _mmb_body(a_ref, b_ref, bias_ref, o_ref, *, act, trans_b):
    a = a_ref[0].astype(jnp.bfloat16)
    b = b_ref[0].astype(jnp.bfloat16)
    r = jnp.dot(a, b, preferred_element_type=jnp.float32)
    o_ref[0] = act(r + bias_ref[0, 0:1, :]).astype(o_ref.dtype)


def _pick_bn(np_):
    for c in (512, 384, 256, 128):
        if np_ % c == 0:
            return c
    return np_


def _mm(a, b, *, act=_act_linear, bias=None, trans_b=False, bn=None,
        out_dtype=jnp.bfloat16):
    _, m, k = a.shape
    b_batched = b.ndim == 3
    bshape = b.shape[1:] if b_batched else b.shape
    np_ = bshape[0] if trans_b else bshape[1]
    if bn is None:
        bn = _pick_bn(np_)
    grid = (_VIEW, np_ // bn)
    in_specs = [pl.BlockSpec((1, m, k), lambda v, j: (v, 0, 0))]
    bsel = (lambda v: v) if b_batched else (lambda v: 0)
    if not b_batched:
        b = b[None]
    if trans_b:
        in_specs.append(pl.BlockSpec((1, bn, k), lambda v, j: (bsel(v), j, 0)))
    else:
        in_specs.append(pl.BlockSpec((1, k, bn), lambda v, j: (bsel(v), 0, j)))
    args = [a, b]
    if bias is not None:
        bias_batched = bias.ndim == 2
        if not bias_batched:
            bias = bias[None]
        bias = bias[:, None, :]
        bisel = (lambda v: v) if bias_batched else (lambda v: 0)
        in_specs.append(
            pl.BlockSpec((1, 1, bn), lambda v, j: (bisel(v), 0, j)))
        args.append(bias)
        body = functools.partial(_mmb_body, act=act, trans_b=trans_b)
    else:
        body = functools.partial(_mm_body, act=act, trans_b=trans_b)
    return pl.pallas_call(
        body,
        grid=grid,
        in_specs=in_specs,
        out_specs=pl.BlockSpec((1, m, bn), lambda v, j: (v, 0, j)),
        out_shape=jax.ShapeDtypeStruct((_VIEW, m, np_), out_dtype),
    )(*args)


def kernel(xs, enc_w1, enc_b1, enc_w2, enc_b2, enc_wz,
           dec_w0, dec_w1, dec_b1, dec_w2, dec_b2,
           fcm_w, fcm_b, lcm_w, lcm_b):
    f32 = jnp.float32
    g0 = _knn(xs)
    gs, deg = _sym(g0)
    s = _scale(gs, deg)

    sx = _mm(s, xs)
    o1 = _mm(sx, enc_w1, bias=enc_b1, act=_act_tanh, bn=_HID)
    so1 = _mm(s, o1, bn=_HID)
    z = _mm(so1, enc_w2, bias=enc_b2, act=_act_tanh, out_dtype=f32)
    zb = z.astype(jnp.bfloat16)

    zwz = _mm(zb, enc_wz)
    ars = _mm(zwz, zb, trans_b=True, act=_act_sigmoid, out_dtype=f32)

    h = _mm(zb, fcm_w, bias=fcm_b, act=_act_l2n, bn=_HD, out_dtype=f32)
    q = _mm(zb, lcm_w, bias=lcm_b, act=_act_softmax, bn=_CN, out_dtype=f32)

    h1 = _mm(zb, dec_w0, act=_act_tanh)
    sh1 = _mm(s, h1)
    h11 = _mm(sh1, dec_w1, bias=dec_b1, act=_act_tanh, bn=_HID)
    sh11 = _mm(s, h11, bn=_HID)
    xr = _mm(sh11, dec_w2, bias=dec_b2, act=_act_tanh, out_dtype=f32)

    return h, q, xr, z, gs, ars
